# baseline (speedup 1.0000x reference)
import functools

import jax
import jax.numpy as jnp
from jax import lax
from jax.experimental import pallas as pl
from jax.experimental.pallas import tpu as pltpu
from jax.experimental.pallas import tpu_sc as plsc

_NC = 2
_NS = 16
_NW = _NC * _NS
_L = 16

_LR = 0.01
_GAMMA = 0.95
_CAPD = 2048
_ZCH = 2048
_NCH = 128


def _count_scatter_add(C, ones, jv):
    pltpu.sync_copy(ones, C.at[jv], add=True)


def _lane_iota():
    return lax.iota(jnp.int32, _L)


def _extract_lane(vec, lane, sentinel):
    return jnp.max(jnp.where(_lane_iota() == lane, vec, sentinel))


@functools.cache
def _build(B, S, A):
    bpw = B // _NW
    bpt = B // _NS
    nblk = bpw // _L
    SB = (S + 1) // 2
    nzch = -(-SB // _ZCH)
    CZ = nzch * _ZCH
    mesh = plsc.VectorSubcoreMesh(
        core_axis_name="c", subcore_axis_name="s",
        num_cores=_NC, num_subcores=_NS)
    cparams = pltpu.CompilerParams(
        use_tc_tiling_on_sc=False, needs_layout_passes=False)

    @functools.partial(
        pl.kernel,
        out_type=jax.ShapeDtypeStruct((B,), jnp.int32),
        mesh=mesh,
        compiler_params=cparams,
        scratch_types=[
            pltpu.VMEM((bpt,), jnp.int32),
            pltpu.VMEM((bpt,), jnp.int32),
            pltpu.VMEM((bpw,), jnp.int32),
            pltpu.VMEM((bpw,), jnp.float32),
            pltpu.VMEM((bpt,), jnp.int32),
            pltpu.VMEM((_ZCH,), jnp.int32),
            pltpu.VMEM((_CAPD,), jnp.int32),
            pltpu.VMEM((_CAPD,), jnp.int32),
            pltpu.VMEM((_CAPD,), jnp.int32),
            pltpu.VMEM((_CAPD,), jnp.float32),
            pltpu.VMEM((_L,), jnp.int32),
            pltpu.VMEM((_ZCH,), jnp.int32),
            pltpu.VMEM((bpw, A), jnp.float32),
            pltpu.VMEM((2, _NCH, A), jnp.float32),
            pltpu.VMEM((A,), jnp.float32),
            pltpu.VMEM((_L,), jnp.float32),
            pltpu.VMEM((_NS, _L), jnp.float32),
            pltpu.VMEM((bpw,), jnp.int32),
            pltpu.VMEM_SHARED((CZ,), jnp.int32),
            pltpu.VMEM_SHARED((B,), jnp.int32),
            pltpu.VMEM_SHARED((B,), jnp.int32),
            pltpu.VMEM_SHARED((B,), jnp.int32),
            pltpu.VMEM_SHARED((B,), jnp.float32),
            pltpu.VMEM_SHARED((_NS, _L), jnp.float32),
            pltpu.SemaphoreType.DMA,
            pltpu.SemaphoreType.DMA,
            pltpu.SemaphoreType.DMA,
        ],
    )
    def _qstep(st_hbm, ac_hbm, rw_hbm, ns_hbm, qt_hbm, out_hbm,
               st_own, nx_own, ac_own, rw_own, cnt_own, cbuf, dj, djs, dja,
               djr, ones, zer, G, nbuf, rowb, accv, pm, outb,
               C, Call, st_sh, ac_sh, rw_sh, pmax_sh,
               semg, sema, semb):
        cid = lax.axis_index("c")
        sid = lax.axis_index("s")
        wid = sid * _NC + cid
        base = wid * bpw
        woff = cid * bpw
        lane = _lane_iota()
        zeros16 = jnp.zeros((_L,), jnp.int32)

        pltpu.sync_copy(st_hbm.at[pl.ds(sid * bpt, bpt)], st_own)
        pltpu.sync_copy(ns_hbm.at[pl.ds(sid * bpt, bpt)], nx_own)
        pltpu.sync_copy(ac_hbm.at[pl.ds(base, bpw)], ac_own)
        pltpu.sync_copy(rw_hbm.at[pl.ds(base, bpw)], rw_own)
        pltpu.sync_copy(st_hbm.at[pl.ds(sid * bpt, bpt)],
                        st_sh.at[pl.ds(sid * bpt, bpt)])
        pltpu.sync_copy(ac_hbm.at[pl.ds(sid * bpt, bpt)],
                        ac_sh.at[pl.ds(sid * bpt, bpt)])
        pltpu.sync_copy(rw_hbm.at[pl.ds(sid * bpt, bpt)],
                        rw_sh.at[pl.ds(sid * bpt, bpt)])
        gdescs = []
        for p in range(bpw // _L):
            jv = st_own[pl.ds(woff + p * _L, _L)]
            gdescs.append(pltpu.async_copy(
                qt_hbm.at[jv], G.at[pl.ds(p * _L, _L)], semg))

        def zfill(k, _):
            zer[pl.ds(k * _L, _L)] = zeros16
            return 0
        lax.fori_loop(0, _ZCH // _L, zfill, 0)
        ones[...] = jnp.ones((_L,), jnp.int32)
        for k in range(-(-nzch // _NS)):
            coff = (sid + k * _NS) * _ZCH

            @pl.when(coff < CZ)
            def _zc(coff=coff):
                pltpu.sync_copy(zer, C.at[pl.ds(coff, _ZCH)])

        nchk = bpt // _NCH

        def nfire(c):
            sem = sema if c % 2 == 0 else semb
            ds_ = []
            for q in range(_NCH // _L):
                jv = nx_own[pl.ds(c * _NCH + q * _L, _L)]
                ds_.append(pltpu.async_copy(
                    qt_hbm.at[jv], nbuf.at[c % 2, pl.ds(q * _L, _L)], sem))
            return ds_
        ndescs = {0: nfire(0)}
        acc = jnp.full((_L,), -jnp.inf, jnp.float32)
        for c in range(nchk):
            if c + 1 < nchk:
                ndescs[c + 1] = nfire(c + 1)
            for d in ndescs.pop(c):
                d.wait()

            def redc(r, acc, c=c):
                for cc in range(A // _L):
                    acc = jnp.maximum(acc, nbuf[c % 2, r, pl.ds(cc * _L, _L)])
                return acc
            acc = lax.fori_loop(0, _NCH, redc, acc)
        accv[...] = acc
        pltpu.sync_copy(accv, pmax_sh.at[sid])
        plsc.subcore_barrier()
        pltpu.sync_copy(pmax_sh, pm)
        accm = pm[0, :]
        for r in range(1, _NS):
            accm = jnp.maximum(accm, pm[r, :])
        M = jnp.max(accm)

        def cadd(k, _):
            jv = st_own[pl.ds(k * _L, _L)] >> 1
            _count_scatter_add(C, ones, jv)
            return 0
        lax.fori_loop(0, bpt // _L, cadd, 0)
        plsc.subcore_barrier()

        def cget(k, _):
            jv = st_own[pl.ds(k * _L, _L)] >> 1
            pltpu.sync_copy(C.at[jv], cnt_own.at[pl.ds(k * _L, _L)])
            return 0
        lax.fori_loop(0, bpt // _L, cget, 0)
        pltpu.sync_copy(cnt_own, Call.at[pl.ds(sid * bpt, bpt)])
        plsc.subcore_barrier()

        def dsuper(cb, nd):
            pltpu.sync_copy(Call.at[pl.ds(cb * _ZCH, _ZCH)], cbuf)

            def dbuild(k, nd):
                cv = cbuf[pl.ds(k * _L, _L)]
                m = cv > 1
                cs = plsc.cumsum(m.astype(jnp.int32))
                pos = nd + cs - 1
                jv = cb * _ZCH + k * _L + lane
                plsc.store_scatter(dj, [pos], jv, mask=m & (pos < _CAPD))
                return nd + jnp.max(cs)
            return lax.fori_loop(0, _ZCH // _L, dbuild, nd)
        nd = lax.fori_loop(0, B // _ZCH, dsuper, jnp.int32(0))

        for d in gdescs:
            d.wait()

        c_old = jnp.float32(1.0 - _LR)
        c_lr = jnp.float32(_LR)
        c_g = jnp.float32(_GAMMA)

        def apply_updates(i_loc, s_i):
            for c in range(A // _L):
                rowb[pl.ds(c * _L, _L)] = G[i_loc, pl.ds(c * _L, _L)]
            iv = jnp.zeros((_L,), jnp.int32) + i_loc

            def chunk(d, _):
                off = d * _L
                mlane = lane < (nd - off)
                sv2 = djs[pl.ds(off, _L)]
                m2 = mlane & (sv2 == s_i)
                av2 = dja[pl.ds(off, _L)]
                rv2 = djr[pl.ds(off, _L)]
                basev = plsc.load_gather(rowb, [av2])
                val = c_old * basev + c_lr * (rv2 + c_g * M)
                plsc.store_scatter(G, [iv, av2], val, mask=m2)
                return 0
            lax.fori_loop(0, (nd + _L - 1) // _L, chunk, 0)

        def dense_contested():
            def dgat(d, _):
                off = d * _L
                jraw = dj[pl.ds(off, _L)]
                jv = jnp.where(lane < (nd - off), jraw, 0)
                pltpu.sync_copy(st_sh.at[jv], djs.at[pl.ds(off, _L)])
                pltpu.sync_copy(ac_sh.at[jv], dja.at[pl.ds(off, _L)])
                pltpu.sync_copy(rw_sh.at[jv], djr.at[pl.ds(off, _L)])
                return 0
            lax.fori_loop(0, (nd + _L - 1) // _L, dgat, 0)

            def outer(d, _):
                off = d * _L
                mlane = lane < (nd - off)
                jraw = dj[pl.ds(off, _L)]
                jv = jnp.where(mlane, jraw, -1)
                mine = mlane & (jv >= base) & (jv < base + bpw)
                mine32 = mine.astype(jnp.int32)

                @pl.when(jnp.max(mine32) > 0)
                def _have():
                    sv = djs[pl.ds(off, _L)]
                    for l in range(_L):
                        @pl.when(_extract_lane(mine32, l, 0) > 0)
                        def _row(l=l, jv=jv, sv=sv):
                            j_l = _extract_lane(jv, l, -1)
                            s_i = _extract_lane(sv, l, -1)
                            apply_updates(j_l - base, s_i)
                return 0
            lax.fori_loop(0, (nd + _L - 1) // _L, outer, 0)

        def brute_contested():
            def blk(b, _):
                cv = cnt_own[pl.ds(woff + b * _L, _L)]

                @pl.when(jnp.max(cv) > 1)
                def _blk():
                    sv = st_own[pl.ds(woff + b * _L, _L)]
                    for l in range(_L):
                        @pl.when(_extract_lane(cv, l, 0) > 1)
                        def _row(l=l, sv=sv):
                            s_i = _extract_lane(sv, l, -1)
                            i_loc = b * _L + l
                            for c in range(A // _L):
                                rowb[pl.ds(c * _L, _L)] = (
                                    G[i_loc, pl.ds(c * _L, _L)])
                            iv = jnp.zeros((_L,), jnp.int32) + i_loc

                            def sup(s9, _):
                                pltpu.sync_copy(
                                    st_sh.at[pl.ds(s9 * _ZCH, _ZCH)], djs)
                                pltpu.sync_copy(
                                    ac_sh.at[pl.ds(s9 * _ZCH, _ZCH)], dja)
                                pltpu.sync_copy(
                                    rw_sh.at[pl.ds(s9 * _ZCH, _ZCH)], djr)

                                def inner(k, _):
                                    sv2 = djs[pl.ds(k * _L, _L)]
                                    m2 = sv2 == s_i
                                    av2 = dja[pl.ds(k * _L, _L)]
                                    rv2 = djr[pl.ds(k * _L, _L)]
                                    basev = plsc.load_gather(rowb, [av2])
                                    val = (c_old * basev
                                           + c_lr * (rv2 + c_g * M))
                                    plsc.store_scatter(G, [iv, av2], val,
                                                       mask=m2)
                                    return 0
                                lax.fori_loop(0, _ZCH // _L, inner, 0)
                                return 0
                            lax.fori_loop(0, B // _ZCH, sup, 0)
                return 0
            lax.fori_loop(0, nblk, blk, 0)

        lax.cond(nd <= _CAPD, dense_contested, brute_contested)

        for b in range(nblk):
            iv = b * _L + lane
            av = ac_own[pl.ds(b * _L, _L)]
            rv = rw_own[pl.ds(b * _L, _L)]
            cv = cnt_own[pl.ds(woff + b * _L, _L)]
            m1 = cv == 1
            old = plsc.load_gather(G, [iv, av])
            val = c_old * old + c_lr * (rv + c_g * M)
            plsc.store_scatter(G, [iv, av], val, mask=m1)

        for b in range(nblk):
            iv = b * _L + lane

            def amax(a, carry):
                m, am = carry
                v = plsc.load_gather(G, [iv, zeros16 + a])
                upd = v > m
                return jnp.maximum(m, v), jnp.where(upd, a, am)
            _, am = lax.fori_loop(
                0, A, amax,
                (jnp.full((_L,), -jnp.inf, jnp.float32), zeros16))
            outb[pl.ds(b * _L, _L)] = am
        pltpu.sync_copy(outb, out_hbm.at[pl.ds(base, bpw)])

    return _qstep


def kernel(state, action, reward, next_state, q_table):
    B = state.shape[0]
    S, A = q_table.shape
    qstep = _build(B, S, A)
    return qstep(state, action, reward, next_state, q_table)


# speedup vs baseline: 1.0279x; 1.0279x over previous
import functools

import jax
import jax.numpy as jnp
from jax import lax
from jax.experimental import pallas as pl
from jax.experimental.pallas import tpu as pltpu
from jax.experimental.pallas import tpu_sc as plsc

_NC = 2
_NS = 16
_NW = _NC * _NS
_L = 16

_LR = 0.01
_GAMMA = 0.95
_CAPD = 2048
_ZCH = 2048
_NCH = 128


def _count_scatter_add(C, ones128, stb, r):
    pltpu.sync_copy(ones128, C.at[stb.at[r]], add=True)


def _count_gather(C, stb, r, cnt_own, sem):
    return pltpu.async_copy(C.at[stb.at[r]], cnt_own.at[pl.ds(r * 128, 128)],
                            sem)


def _lane_iota():
    return lax.iota(jnp.int32, _L)


def _extract_lane(vec, lane, sentinel):
    return jnp.max(jnp.where(_lane_iota() == lane, vec, sentinel))


@functools.cache
def _build(B, S, A):
    bpw = B // _NW
    bpt = B // _NS
    nblk = bpw // _L
    SB = (S + 1) // 2
    nzch = -(-SB // _ZCH)
    CZ = nzch * _ZCH
    mesh = plsc.VectorSubcoreMesh(
        core_axis_name="c", subcore_axis_name="s",
        num_cores=_NC, num_subcores=_NS)
    cparams = pltpu.CompilerParams(
        use_tc_tiling_on_sc=False, needs_layout_passes=False)

    @functools.partial(
        pl.kernel,
        out_type=jax.ShapeDtypeStruct((B,), jnp.int32),
        mesh=mesh,
        compiler_params=cparams,
        scratch_types=[
            pltpu.VMEM((bpt,), jnp.int32),
            pltpu.VMEM((bpt,), jnp.int32),
            pltpu.VMEM((bpw,), jnp.int32),
            pltpu.VMEM((bpw,), jnp.float32),
            pltpu.VMEM((bpt,), jnp.int32),
            pltpu.VMEM((_ZCH,), jnp.int32),
            pltpu.VMEM((_CAPD,), jnp.int32),
            pltpu.VMEM((_CAPD,), jnp.int32),
            pltpu.VMEM((_CAPD,), jnp.int32),
            pltpu.VMEM((_CAPD,), jnp.float32),
            pltpu.VMEM((128,), jnp.int32),
            pltpu.VMEM((bpt // 128, 128), jnp.int32),
            pltpu.VMEM((_ZCH,), jnp.int32),
            pltpu.VMEM((bpw, A), jnp.float32),
            pltpu.VMEM((2, _NCH, A), jnp.float32),
            pltpu.VMEM((A,), jnp.float32),
            pltpu.VMEM((_L,), jnp.float32),
            pltpu.VMEM((_NS, _L), jnp.float32),
            pltpu.VMEM((bpw,), jnp.int32),
            pltpu.VMEM_SHARED((CZ,), jnp.int32),
            pltpu.VMEM_SHARED((B,), jnp.int32),
            pltpu.VMEM_SHARED((B,), jnp.int32),
            pltpu.VMEM_SHARED((B,), jnp.int32),
            pltpu.VMEM_SHARED((B,), jnp.float32),
            pltpu.VMEM_SHARED((_NS, _L), jnp.float32),
            pltpu.SemaphoreType.DMA,
            pltpu.SemaphoreType.DMA,
            pltpu.SemaphoreType.DMA,
        ],
    )
    def _qstep(st_hbm, ac_hbm, rw_hbm, ns_hbm, qt_hbm, out_hbm,
               st_own, nx_own, ac_own, rw_own, cnt_own, cbuf, dj, djs, dja,
               djr, ones128, stb, zer, G, nbuf, rowb, accv, pm, outb,
               C, Call, st_sh, ac_sh, rw_sh, pmax_sh,
               semg, sema, semb):
        cid = lax.axis_index("c")
        sid = lax.axis_index("s")
        wid = sid * _NC + cid
        base = wid * bpw
        woff = cid * bpw
        lane = _lane_iota()
        zeros16 = jnp.zeros((_L,), jnp.int32)

        pltpu.sync_copy(st_hbm.at[pl.ds(sid * bpt, bpt)], st_own)
        pltpu.sync_copy(ns_hbm.at[pl.ds(sid * bpt, bpt)], nx_own)
        pltpu.sync_copy(ac_hbm.at[pl.ds(base, bpw)], ac_own)
        pltpu.sync_copy(rw_hbm.at[pl.ds(base, bpw)], rw_own)
        pltpu.sync_copy(st_hbm.at[pl.ds(sid * bpt, bpt)],
                        st_sh.at[pl.ds(sid * bpt, bpt)])
        pltpu.sync_copy(ac_hbm.at[pl.ds(sid * bpt, bpt)],
                        ac_sh.at[pl.ds(sid * bpt, bpt)])
        pltpu.sync_copy(rw_hbm.at[pl.ds(sid * bpt, bpt)],
                        rw_sh.at[pl.ds(sid * bpt, bpt)])
        gdescs = []
        for p in range(bpw // _L):
            jv = st_own[pl.ds(woff + p * _L, _L)]
            gdescs.append(pltpu.async_copy(
                qt_hbm.at[jv], G.at[pl.ds(p * _L, _L)], semg))

        def zfill(k, _):
            for u in range(4):
                zer[pl.ds((k * 4 + u) * _L, _L)] = zeros16
            return 0
        lax.fori_loop(0, _ZCH // _L // 4, zfill, 0)
        for k in range(128 // _L):
            ones128[pl.ds(k * _L, _L)] = jnp.ones((_L,), jnp.int32)
        for k in range(bpt // _L):
            stb[k // 8, pl.ds((k % 8) * _L, _L)] = (
                st_own[pl.ds(k * _L, _L)] >> 1)
        for k in range(-(-nzch // _NS)):
            coff = (sid + k * _NS) * _ZCH

            @pl.when(coff < CZ)
            def _zc(coff=coff):
                pltpu.sync_copy(zer, C.at[pl.ds(coff, _ZCH)])

        nchk = bpt // _NCH

        def nfire(c):
            sem = sema if c % 2 == 0 else semb
            ds_ = []
            for q in range(_NCH // _L):
                jv = nx_own[pl.ds(c * _NCH + q * _L, _L)]
                ds_.append(pltpu.async_copy(
                    qt_hbm.at[jv], nbuf.at[c % 2, pl.ds(q * _L, _L)], sem))
            return ds_
        ndescs = {0: nfire(0)}
        acc = jnp.full((_L,), -jnp.inf, jnp.float32)
        for c in range(nchk):
            if c + 1 < nchk:
                ndescs[c + 1] = nfire(c + 1)
            for d in ndescs.pop(c):
                d.wait()

            def redc(r, acc, c=c):
                for u in range(4):
                    for cc in range(A // _L):
                        acc = jnp.maximum(
                            acc, nbuf[c % 2, r * 4 + u, pl.ds(cc * _L, _L)])
                return acc
            acc = lax.fori_loop(0, _NCH // 4, redc, acc)
        accv[...] = acc
        pltpu.sync_copy(accv, pmax_sh.at[sid])
        plsc.subcore_barrier()
        pltpu.sync_copy(pmax_sh, pm)
        accm = pm[0, :]
        for r in range(1, _NS):
            accm = jnp.maximum(accm, pm[r, :])
        M = jnp.max(accm)

        for r in range(bpt // 128):
            _count_scatter_add(C, ones128, stb, r)
        plsc.subcore_barrier()

        cdescs = [_count_gather(C, stb, r, cnt_own, sema)
                  for r in range(bpt // 128)]
        for d in cdescs:
            d.wait()
        pltpu.sync_copy(cnt_own, Call.at[pl.ds(sid * bpt, bpt)])
        plsc.subcore_barrier()

        def dsuper(cb, nd):
            pltpu.sync_copy(Call.at[pl.ds(cb * _ZCH, _ZCH)], cbuf)

            def dbuild(k, nd):
                for u in range(4):
                    cv = cbuf[pl.ds((k * 4 + u) * _L, _L)]
                    m = cv > 1
                    cs = plsc.cumsum(m.astype(jnp.int32))
                    pos = nd + cs - 1
                    jv = cb * _ZCH + (k * 4 + u) * _L + lane
                    plsc.store_scatter(dj, [pos], jv, mask=m & (pos < _CAPD))
                    nd = nd + jnp.max(cs)
                return nd
            return lax.fori_loop(0, _ZCH // _L // 4, dbuild, nd)
        nd = lax.fori_loop(0, B // _ZCH, dsuper, jnp.int32(0))

        for d in gdescs:
            d.wait()

        c_old = jnp.float32(1.0 - _LR)
        c_lr = jnp.float32(_LR)
        c_g = jnp.float32(_GAMMA)

        def apply_updates(i_loc, s_i):
            for c in range(A // _L):
                rowb[pl.ds(c * _L, _L)] = G[i_loc, pl.ds(c * _L, _L)]
            iv = jnp.zeros((_L,), jnp.int32) + i_loc

            def chunk(d, _):
                off = d * _L
                mlane = lane < (nd - off)
                sv2 = djs[pl.ds(off, _L)]
                m2 = mlane & (sv2 == s_i)
                av2 = dja[pl.ds(off, _L)]
                rv2 = djr[pl.ds(off, _L)]
                basev = plsc.load_gather(rowb, [av2])
                val = c_old * basev + c_lr * (rv2 + c_g * M)
                plsc.store_scatter(G, [iv, av2], val, mask=m2)
                return 0
            lax.fori_loop(0, (nd + _L - 1) // _L, chunk, 0)

        def dense_contested():
            def dgat(d, _):
                off = d * _L
                jraw = dj[pl.ds(off, _L)]
                jv = jnp.where(lane < (nd - off), jraw, 0)
                pltpu.sync_copy(st_sh.at[jv], djs.at[pl.ds(off, _L)])
                pltpu.sync_copy(ac_sh.at[jv], dja.at[pl.ds(off, _L)])
                pltpu.sync_copy(rw_sh.at[jv], djr.at[pl.ds(off, _L)])
                return 0
            lax.fori_loop(0, (nd + _L - 1) // _L, dgat, 0)

            def outer(d, _):
                off = d * _L
                mlane = lane < (nd - off)
                jraw = dj[pl.ds(off, _L)]
                jv = jnp.where(mlane, jraw, -1)
                mine = mlane & (jv >= base) & (jv < base + bpw)
                mine32 = mine.astype(jnp.int32)

                @pl.when(jnp.max(mine32) > 0)
                def _have():
                    sv = djs[pl.ds(off, _L)]
                    for l in range(_L):
                        @pl.when(_extract_lane(mine32, l, 0) > 0)
                        def _row(l=l, jv=jv, sv=sv):
                            j_l = _extract_lane(jv, l, -1)
                            s_i = _extract_lane(sv, l, -1)
                            apply_updates(j_l - base, s_i)
                return 0
            lax.fori_loop(0, (nd + _L - 1) // _L, outer, 0)

        def brute_contested():
            def blk(b, _):
                cv = cnt_own[pl.ds(woff + b * _L, _L)]

                @pl.when(jnp.max(cv) > 1)
                def _blk():
                    sv = st_own[pl.ds(woff + b * _L, _L)]
                    for l in range(_L):
                        @pl.when(_extract_lane(cv, l, 0) > 1)
                        def _row(l=l, sv=sv):
                            s_i = _extract_lane(sv, l, -1)
                            i_loc = b * _L + l
                            for c in range(A // _L):
                                rowb[pl.ds(c * _L, _L)] = (
                                    G[i_loc, pl.ds(c * _L, _L)])
                            iv = jnp.zeros((_L,), jnp.int32) + i_loc

                            def sup(s9, _):
                                pltpu.sync_copy(
                                    st_sh.at[pl.ds(s9 * _ZCH, _ZCH)], djs)
                                pltpu.sync_copy(
                                    ac_sh.at[pl.ds(s9 * _ZCH, _ZCH)], dja)
                                pltpu.sync_copy(
                                    rw_sh.at[pl.ds(s9 * _ZCH, _ZCH)], djr)

                                def inner(k, _):
                                    sv2 = djs[pl.ds(k * _L, _L)]
                                    m2 = sv2 == s_i
                                    av2 = dja[pl.ds(k * _L, _L)]
                                    rv2 = djr[pl.ds(k * _L, _L)]
                                    basev = plsc.load_gather(rowb, [av2])
                                    val = (c_old * basev
                                           + c_lr * (rv2 + c_g * M))
                                    plsc.store_scatter(G, [iv, av2], val,
                                                       mask=m2)
                                    return 0
                                lax.fori_loop(0, _ZCH // _L, inner, 0)
                                return 0
                            lax.fori_loop(0, B // _ZCH, sup, 0)
                return 0
            lax.fori_loop(0, nblk, blk, 0)

        lax.cond(nd <= _CAPD, dense_contested, brute_contested)

        for b in range(nblk):
            iv = b * _L + lane
            av = ac_own[pl.ds(b * _L, _L)]
            rv = rw_own[pl.ds(b * _L, _L)]
            cv = cnt_own[pl.ds(woff + b * _L, _L)]
            m1 = cv == 1
            old = plsc.load_gather(G, [iv, av])
            val = c_old * old + c_lr * (rv + c_g * M)
            plsc.store_scatter(G, [iv, av], val, mask=m1)

        for b in range(nblk):
            iv = b * _L + lane

            def amax(a4, carry):
                m, am = carry
                for u in range(4):
                    a = a4 * 4 + u
                    v = plsc.load_gather(G, [iv, zeros16 + a])
                    upd = v > m
                    m, am = jnp.maximum(m, v), jnp.where(upd, a, am)
                return m, am
            _, am = lax.fori_loop(
                0, A // 4, amax,
                (jnp.full((_L,), -jnp.inf, jnp.float32), zeros16))
            outb[pl.ds(b * _L, _L)] = am
        pltpu.sync_copy(outb, out_hbm.at[pl.ds(base, bpw)])

    return _qstep


def kernel(state, action, reward, next_state, q_table):
    B = state.shape[0]
    S, A = q_table.shape
    qstep = _build(B, S, A)
    return qstep(state, action, reward, next_state, q_table)
